# baseline (speedup 1.0000x reference)
import functools

import jax
import jax.numpy as jnp
from jax import lax
from jax.experimental import pallas as pl
from jax.experimental.pallas import tpu as pltpu
from jax.experimental.pallas import tpu_sc as plsc

N_NODES = 10000
N_EDGES = 160000
D_FEAT = 256

NC = 2
NS = 16
LANES = 16

CHUNK = 80
NCHUNK_TOT = 125
N_FAST = 63
N_SLOW = NCHUNK_TOT - N_FAST
FAST_CORE = 0

TPAD = LANES + 1


def _sc_body(h_hbm, ei_hbm, out_hbm,
             src_v, dst_v, u0_v, v0_v, u1_v, v1_v, out_v, tr_v,
             sem0, sem1):
    cid = lax.axis_index("c")
    sid = lax.axis_index("s")

    lane = jax.lax.iota(jnp.int32, LANES)

    def issue(lc, u_v, v_v, sem):
        pltpu.async_copy(
            h_hbm.at[src_v.at[pl.ds(lc * CHUNK, CHUNK)]], u_v, sem)
        pltpu.async_copy(
            h_hbm.at[dst_v.at[pl.ds(lc * CHUNK, CHUNK)]], v_v, sem)

    def drain(u_v, v_v, sem):
        pltpu.make_async_copy(h_hbm.at[pl.ds(0, CHUNK)], u_v, sem).wait()
        pltpu.make_async_copy(h_hbm.at[pl.ds(0, CHUNK)], v_v, sem).wait()

    def compute(lc, u_v, v_v):
        def group_body(g, carry2):
            def edge_body(t, carry3):
                e = g * LANES + t
                nj = D_FEAT // (2 * LANES)

                def bf(ref, j):
                    return plsc.bitcast(ref[e, pl.ds(j * LANES, LANES)],
                                        jnp.bfloat16)

                accs = [bf(u_v, j) * bf(v_v, j) for j in range(2)]
                for j in range(2, nj):
                    accs[j % 2] = accs[j % 2] + bf(u_v, j) * bf(v_v, j)
                acc = accs[0] + accs[1]
                lo, hi = plsc.unpack(acc, format=plsc.PackFormat.INTERLEAVED)
                tr_v[pl.ds(t * TPAD, LANES)] = lo + hi
                return carry3

            lax.fori_loop(0, LANES, edge_body, 0, unroll=4)

            res = plsc.load_gather(tr_v, [lane * TPAD])
            for j in range(1, LANES):
                res = res + plsc.load_gather(tr_v, [lane * TPAD + j])
            out_v[pl.ds(lc * CHUNK + g * LANES, LANES)] = res
            return carry2

        lax.fori_loop(0, CHUNK // LANES, group_body, 0, unroll=False)

    def run(cstart, n):
        ne = n * CHUNK
        base = cstart * CHUNK
        pltpu.sync_copy(ei_hbm.at[pl.ds(base, ne)], src_v.at[pl.ds(0, ne)])
        pltpu.sync_copy(ei_hbm.at[pl.ds(N_EDGES + base, ne)],
                        dst_v.at[pl.ds(0, ne)])

        half = n // 2
        issue(0, u0_v, v0_v, sem0)

        def pipe_body(i, carry):
            c0 = 2 * i
            issue(c0 + 1, u1_v, v1_v, sem1)
            drain(u0_v, v0_v, sem0)
            compute(c0, u0_v, v0_v)

            if n % 2 == 1:
                issue(c0 + 2, u0_v, v0_v, sem0)
            else:
                @pl.when(i < half - 1)
                def _():
                    issue(c0 + 2, u0_v, v0_v, sem0)

            drain(u1_v, v1_v, sem1)
            compute(c0 + 1, u1_v, v1_v)
            return carry

        lax.fori_loop(0, half, pipe_body, 0, unroll=False)

        if n % 2 == 1:
            drain(u0_v, v0_v, sem0)
            compute(n - 1, u0_v, v0_v)

        pltpu.sync_copy(
            out_v.at[pl.ds(0, n * CHUNK)],
            out_hbm.at[pl.ds(cstart * CHUNK, n * CHUNK)],
        )

    @pl.when(cid == FAST_CORE)
    def _():
        run(sid * N_FAST, N_FAST)

    @pl.when(cid == 1 - FAST_CORE)
    def _():
        run(NS * N_FAST + sid * N_SLOW, N_SLOW)


PK_ROWS_W = 312
PK_RCHUNK = 24
PK_TAIL = N_NODES - (NC * NS - 1) * PK_ROWS_W - PK_ROWS_W


def _pack_chunk(in_v, out_v, nrows):
    def row_body(r, carry):
        for j in range(D_FEAT // (2 * LANES)):
            a = in_v[r, pl.ds(j * 2 * LANES, LANES)]
            b = in_v[r, pl.ds(j * 2 * LANES + LANES, LANES)]
            pk = plsc.pack(a, b, format=plsc.PackFormat.INTERLEAVED)
            out_v[r, pl.ds(j * LANES, LANES)] = plsc.bitcast(pk, jnp.int32)
        return carry

    lax.fori_loop(0, nrows, row_body, 0, unroll=2)


def _pack_body(h_hbm, out_hbm, in0_v, in1_v, out0_v, out1_v, isem, osem):
    wid = lax.axis_index("s") * NC + lax.axis_index("c")
    r0 = wid * PK_ROWS_W
    in_bufs = (in0_v, in1_v)
    out_bufs = (out0_v, out1_v)
    nck = PK_ROWS_W // PK_RCHUNK

    pltpu.async_copy(h_hbm.at[pl.ds(r0, PK_RCHUNK)], in0_v, isem)
    for k in range(nck):
        if k + 1 < nck:
            pltpu.async_copy(
                h_hbm.at[pl.ds(r0 + (k + 1) * PK_RCHUNK, PK_RCHUNK)],
                in_bufs[(k + 1) % 2], isem)
        pltpu.make_async_copy(
            h_hbm.at[pl.ds(0, PK_RCHUNK)], in_bufs[k % 2], isem).wait()
        if k >= 2:
            pltpu.make_async_copy(
                out_hbm.at[pl.ds(0, PK_RCHUNK)], out_bufs[k % 2], osem).wait()
        _pack_chunk(in_bufs[k % 2], out_bufs[k % 2], PK_RCHUNK)
        pltpu.async_copy(
            out_bufs[k % 2],
            out_hbm.at[pl.ds(r0 + k * PK_RCHUNK, PK_RCHUNK)], osem)
    for k in (nck - 2, nck - 1):
        pltpu.make_async_copy(
            out_hbm.at[pl.ds(0, PK_RCHUNK)], out_bufs[k % 2], osem).wait()

    @pl.when(wid == NC * NS - 1)
    def _():
        tr0 = NC * NS * PK_ROWS_W
        pltpu.sync_copy(h_hbm.at[pl.ds(tr0, PK_TAIL)],
                        in0_v.at[pl.ds(0, PK_TAIL)])
        _pack_chunk(in0_v, out0_v, PK_TAIL)
        pltpu.sync_copy(out0_v.at[pl.ds(0, PK_TAIL)],
                        out_hbm.at[pl.ds(tr0, PK_TAIL)])


@jax.jit
def _pack_h(h):
    mesh = plsc.VectorSubcoreMesh(core_axis_name="c", subcore_axis_name="s")
    kern = functools.partial(
        pl.kernel,
        mesh=mesh,
        out_type=jax.ShapeDtypeStruct((N_NODES, D_FEAT // 2), jnp.int32),
        scratch_types=[
            pltpu.VMEM((PK_RCHUNK, D_FEAT), jnp.float32),
            pltpu.VMEM((PK_RCHUNK, D_FEAT), jnp.float32),
            pltpu.VMEM((PK_RCHUNK, D_FEAT // 2), jnp.int32),
            pltpu.VMEM((PK_RCHUNK, D_FEAT // 2), jnp.int32),
            pltpu.SemaphoreType.DMA,
            pltpu.SemaphoreType.DMA,
        ],
        compiler_params=pltpu.CompilerParams(needs_layout_passes=False),
    )(_pack_body)
    return kern(h)


@jax.jit
def _dot_scores(h, ei):
    mesh = plsc.VectorSubcoreMesh(core_axis_name="c", subcore_axis_name="s")
    kern = functools.partial(
        pl.kernel,
        mesh=mesh,
        out_type=jax.ShapeDtypeStruct((N_EDGES,), jnp.float32),
        scratch_types=[
            pltpu.VMEM((N_FAST * CHUNK,), jnp.int32),
            pltpu.VMEM((N_FAST * CHUNK,), jnp.int32),
            pltpu.VMEM((CHUNK, D_FEAT // 2), jnp.int32),
            pltpu.VMEM((CHUNK, D_FEAT // 2), jnp.int32),
            pltpu.VMEM((CHUNK, D_FEAT // 2), jnp.int32),
            pltpu.VMEM((CHUNK, D_FEAT // 2), jnp.int32),
            pltpu.VMEM((N_FAST * CHUNK,), jnp.float32),
            pltpu.VMEM((LANES * TPAD,), jnp.float32),
            pltpu.SemaphoreType.DMA,
            pltpu.SemaphoreType.DMA,
        ],
        compiler_params=pltpu.CompilerParams(needs_layout_passes=False),
    )(_sc_body)
    return kern(h, ei)


def kernel(h, edge_index):
    ei = edge_index.astype(jnp.int32).reshape(2 * N_EDGES)
    h_pk = _pack_h(h)
    return _dot_scores(h_pk, ei)


# speedup vs baseline: 1.0244x; 1.0244x over previous
import functools

import jax
import jax.numpy as jnp
from jax import lax
from jax.experimental import pallas as pl
from jax.experimental.pallas import tpu as pltpu
from jax.experimental.pallas import tpu_sc as plsc

N_NODES = 10000
N_EDGES = 160000
D_FEAT = 256

NC = 2
NS = 16
LANES = 16

CHUNK = 80
NCHUNK_TOT = 125
N_FAST = 63
N_SLOW = NCHUNK_TOT - N_FAST
FAST_CORE = 0

TPAD = LANES + 1


def _sc_body(h_hbm, ei_hbm, out_hbm,
             src_v, dst_v, u0_v, v0_v, u1_v, v1_v, out_v, tr_v,
             sem0, sem1):
    cid = lax.axis_index("c")
    sid = lax.axis_index("s")

    lane = jax.lax.iota(jnp.int32, LANES)

    def issue(lc, u_v, v_v, sem):
        pltpu.async_copy(
            h_hbm.at[src_v.at[pl.ds(lc * CHUNK, CHUNK)]], u_v, sem)
        pltpu.async_copy(
            h_hbm.at[dst_v.at[pl.ds(lc * CHUNK, CHUNK)]], v_v, sem)

    def drain(u_v, v_v, sem):
        pltpu.make_async_copy(h_hbm.at[pl.ds(0, CHUNK)], u_v, sem).wait()
        pltpu.make_async_copy(h_hbm.at[pl.ds(0, CHUNK)], v_v, sem).wait()

    def compute(lc, u_v, v_v):
        def group_body(g, carry2):
            def edge_body(t, carry3):
                e = g * LANES + t
                nj = D_FEAT // (2 * LANES)

                def bf(ref, j):
                    return plsc.bitcast(ref[e, pl.ds(j * LANES, LANES)],
                                        jnp.bfloat16)

                accs = [bf(u_v, j) * bf(v_v, j) for j in range(2)]
                for j in range(2, nj):
                    accs[j % 2] = accs[j % 2] + bf(u_v, j) * bf(v_v, j)
                acc = accs[0] + accs[1]
                lo, hi = plsc.unpack(acc, format=plsc.PackFormat.INTERLEAVED)
                tr_v[pl.ds(t * TPAD, LANES)] = lo + hi
                return carry3

            lax.fori_loop(0, LANES, edge_body, 0, unroll=4)

            res = plsc.load_gather(tr_v, [lane * TPAD])
            for j in range(1, LANES):
                res = res + plsc.load_gather(tr_v, [lane * TPAD + j])
            out_v[pl.ds(lc * CHUNK + g * LANES, LANES)] = res
            return carry2

        lax.fori_loop(0, CHUNK // LANES, group_body, 0, unroll=False)

    def run(cstart, n):
        ne = n * CHUNK
        base = cstart * CHUNK
        pltpu.sync_copy(ei_hbm.at[pl.ds(base, ne)], src_v.at[pl.ds(0, ne)])
        pltpu.sync_copy(ei_hbm.at[pl.ds(N_EDGES + base, ne)],
                        dst_v.at[pl.ds(0, ne)])

        half = n // 2
        issue(0, u0_v, v0_v, sem0)

        def pipe_body(i, carry):
            c0 = 2 * i
            issue(c0 + 1, u1_v, v1_v, sem1)
            drain(u0_v, v0_v, sem0)
            compute(c0, u0_v, v0_v)

            if n % 2 == 1:
                issue(c0 + 2, u0_v, v0_v, sem0)
            else:
                @pl.when(i < half - 1)
                def _():
                    issue(c0 + 2, u0_v, v0_v, sem0)

            drain(u1_v, v1_v, sem1)
            compute(c0 + 1, u1_v, v1_v)
            return carry

        lax.fori_loop(0, half, pipe_body, 0, unroll=False)

        if n % 2 == 1:
            drain(u0_v, v0_v, sem0)
            compute(n - 1, u0_v, v0_v)

        pltpu.sync_copy(
            out_v.at[pl.ds(0, n * CHUNK)],
            out_hbm.at[pl.ds(cstart * CHUNK, n * CHUNK)],
        )

    @pl.when(cid == FAST_CORE)
    def _():
        run(sid * N_FAST, N_FAST)

    @pl.when(cid == 1 - FAST_CORE)
    def _():
        run(NS * N_FAST + sid * N_SLOW, N_SLOW)


PK_ROWS_W = 312
PK_RCHUNK = 104
PK_TAIL = N_NODES - (NC * NS - 1) * PK_ROWS_W - PK_ROWS_W


def _pack_chunk(in_v, out_v, nrows):
    def row_body(r, carry):
        for j in range(D_FEAT // (2 * LANES)):
            a = in_v[r, pl.ds(j * 2 * LANES, LANES)]
            b = in_v[r, pl.ds(j * 2 * LANES + LANES, LANES)]
            pk = plsc.pack(a, b, format=plsc.PackFormat.INTERLEAVED)
            out_v[r, pl.ds(j * LANES, LANES)] = plsc.bitcast(pk, jnp.int32)
        return carry

    lax.fori_loop(0, nrows, row_body, 0, unroll=2)


def _pack_body(h_hbm, out_hbm, in0_v, in1_v, out0_v, out1_v, isem, osem):
    wid = lax.axis_index("s") * NC + lax.axis_index("c")
    r0 = wid * PK_ROWS_W
    in_bufs = (in0_v, in1_v)
    out_bufs = (out0_v, out1_v)
    nck = PK_ROWS_W // PK_RCHUNK

    pltpu.async_copy(h_hbm.at[pl.ds(r0, PK_RCHUNK)], in0_v, isem)
    for k in range(nck):
        if k + 1 < nck:
            pltpu.async_copy(
                h_hbm.at[pl.ds(r0 + (k + 1) * PK_RCHUNK, PK_RCHUNK)],
                in_bufs[(k + 1) % 2], isem)
        pltpu.make_async_copy(
            h_hbm.at[pl.ds(0, PK_RCHUNK)], in_bufs[k % 2], isem).wait()
        if k >= 2:
            pltpu.make_async_copy(
                out_hbm.at[pl.ds(0, PK_RCHUNK)], out_bufs[k % 2], osem).wait()
        _pack_chunk(in_bufs[k % 2], out_bufs[k % 2], PK_RCHUNK)
        pltpu.async_copy(
            out_bufs[k % 2],
            out_hbm.at[pl.ds(r0 + k * PK_RCHUNK, PK_RCHUNK)], osem)
    for k in (nck - 2, nck - 1):
        pltpu.make_async_copy(
            out_hbm.at[pl.ds(0, PK_RCHUNK)], out_bufs[k % 2], osem).wait()

    @pl.when(wid == NC * NS - 1)
    def _():
        tr0 = NC * NS * PK_ROWS_W
        pltpu.sync_copy(h_hbm.at[pl.ds(tr0, PK_TAIL)],
                        in0_v.at[pl.ds(0, PK_TAIL)])
        _pack_chunk(in0_v, out0_v, PK_TAIL)
        pltpu.sync_copy(out0_v.at[pl.ds(0, PK_TAIL)],
                        out_hbm.at[pl.ds(tr0, PK_TAIL)])


@jax.jit
def _pack_h(h):
    mesh = plsc.VectorSubcoreMesh(core_axis_name="c", subcore_axis_name="s")
    kern = functools.partial(
        pl.kernel,
        mesh=mesh,
        out_type=jax.ShapeDtypeStruct((N_NODES, D_FEAT // 2), jnp.int32),
        scratch_types=[
            pltpu.VMEM((PK_RCHUNK, D_FEAT), jnp.float32),
            pltpu.VMEM((PK_RCHUNK, D_FEAT), jnp.float32),
            pltpu.VMEM((PK_RCHUNK, D_FEAT // 2), jnp.int32),
            pltpu.VMEM((PK_RCHUNK, D_FEAT // 2), jnp.int32),
            pltpu.SemaphoreType.DMA,
            pltpu.SemaphoreType.DMA,
        ],
        compiler_params=pltpu.CompilerParams(needs_layout_passes=False),
    )(_pack_body)
    return kern(h)


@jax.jit
def _dot_scores(h, ei):
    mesh = plsc.VectorSubcoreMesh(core_axis_name="c", subcore_axis_name="s")
    kern = functools.partial(
        pl.kernel,
        mesh=mesh,
        out_type=jax.ShapeDtypeStruct((N_EDGES,), jnp.float32),
        scratch_types=[
            pltpu.VMEM((N_FAST * CHUNK,), jnp.int32),
            pltpu.VMEM((N_FAST * CHUNK,), jnp.int32),
            pltpu.VMEM((CHUNK, D_FEAT // 2), jnp.int32),
            pltpu.VMEM((CHUNK, D_FEAT // 2), jnp.int32),
            pltpu.VMEM((CHUNK, D_FEAT // 2), jnp.int32),
            pltpu.VMEM((CHUNK, D_FEAT // 2), jnp.int32),
            pltpu.VMEM((N_FAST * CHUNK,), jnp.float32),
            pltpu.VMEM((LANES * TPAD,), jnp.float32),
            pltpu.SemaphoreType.DMA,
            pltpu.SemaphoreType.DMA,
        ],
        compiler_params=pltpu.CompilerParams(needs_layout_passes=False),
    )(_sc_body)
    return kern(h, ei)


def kernel(h, edge_index):
    ei = edge_index.astype(jnp.int32).reshape(2 * N_EDGES)
    h_pk = _pack_h(h)
    return _dot_scores(h_pk, ei)
